# baseline (speedup 1.0000x reference)
import jax
import jax.numpy as jnp
from jax import lax
from jax.experimental import pallas as pl
from jax.experimental.pallas import tpu as pltpu
from jax.experimental.pallas import tpu_sc as plsc

B, C, H, W = 64, 3, 512, 512
L = 16
NC, NS = 2, 16
NWORK = NC * NS
ROWS_PER_W = B // NWORK
NBUF = 3
HB = 32
CPP = H // HB
CPR = C * CPP
NCHUNK = ROWS_PER_W * CPR
NGROUP = NCHUNK // NBUF


def _sc_body(in_hbm, p_hbm, m_hbm, out_hbm, pm_v, *scratch):
    bufs_in = scratch[:NBUF]
    bufs_out = scratch[NBUF:2 * NBUF]
    sem_in = scratch[2 * NBUF:3 * NBUF]
    sem_out = scratch[3 * NBUF:4 * NBUF]
    sem_pm = scratch[4 * NBUF]

    wid = lax.axis_index("s") * NC + lax.axis_index("c")
    b0 = wid * ROWS_PER_W

    pltpu.async_copy(p_hbm.at[pl.ds(wid * (ROWS_PER_W * L), ROWS_PER_W * L)],
                     pm_v.at[pl.ds(0, ROWS_PER_W * L)], sem_pm)
    pltpu.async_copy(m_hbm.at[pl.ds(wid * (ROWS_PER_W * L), ROWS_PER_W * L)],
                     pm_v.at[pl.ds(ROWS_PER_W * L, ROWS_PER_W * L)], sem_pm)

    def chunk_coords(k):
        r = k // CPR
        rem = k - r * CPR
        c = rem // CPP
        h0 = (rem - c * CPP) * HB
        return b0 + r, r, c, h0

    for b in range(NBUF):
        pltpu.async_copy(in_hbm.at[b0, 0, pl.ds(b * HB, HB), :],
                         bufs_in[b], sem_in[b])

    pltpu.make_async_copy(p_hbm.at[pl.ds(0, ROWS_PER_W * L)],
                          pm_v.at[pl.ds(0, ROWS_PER_W * L)], sem_pm).wait()
    pltpu.make_async_copy(p_hbm.at[pl.ds(0, ROWS_PER_W * L)],
                          pm_v.at[pl.ds(0, ROWS_PER_W * L)], sem_pm).wait()

    addends = []
    for r in range(ROWS_PER_W):
        pvec = pm_v[pl.ds(r * L, L)]
        mvec = pm_v[pl.ds(ROWS_PER_W * L + r * L, L)]
        addends.append(jnp.where(pvec != 0.0, mvec, 0.0))

    def group(g, _):
        for b in range(NBUF):
            k = g * NBUF + b
            bi, r, c, h0 = chunk_coords(k)

            pltpu.make_async_copy(in_hbm.at[bi, c, pl.ds(h0, HB), :],
                                  bufs_in[b], sem_in[b]).wait()

            @pl.when(g >= 1)
            def _():
                pltpu.make_async_copy(
                    bufs_out[b], out_hbm.at[b0, 0, pl.ds(0, HB), :],
                    sem_out[b]).wait()

            addend = jnp.where(r == 0, addends[0], addends[1])

            @plsc.parallel_loop(0, HB)
            def _(i):
                for j in range(W // L):
                    sl = pl.ds(j * L, L)
                    bufs_out[b][i, sl] = bufs_in[b][i, sl] + addend

            pltpu.async_copy(bufs_out[b], out_hbm.at[bi, c, pl.ds(h0, HB), :],
                             sem_out[b])

            @pl.when(g < NGROUP - 1)
            def _():
                bn, _, cn, hn = chunk_coords(k + NBUF)
                pltpu.async_copy(in_hbm.at[bn, cn, pl.ds(hn, HB), :],
                                 bufs_in[b], sem_in[b])
        return 0

    sh = scratch[4 * NBUF + 1]
    sem_sh_in = scratch[4 * NBUF + 2:4 * NBUF + 4]
    sem_sh_out = scratch[4 * NBUF + 4:4 * NBUF + 6]
    sid = lax.axis_index("s")

    def group2(g2, _):
        off = lax.rem(g2, CPP) * HB
        for q in range(2):
            @pl.when(g2 >= 1)
            def _():
                pltpu.make_async_copy(
                    sh.at[sid, q], out_hbm.at[b0, 1, pl.ds(0, HB // 2), :],
                    sem_sh_out[q]).wait()

            @pl.when(g2 < NGROUP - 1)
            def _():
                pltpu.async_copy(in_hbm.at[b0, 0, pl.ds(off, HB // 2), :],
                                 sh.at[sid, q], sem_sh_in[q])
        group(g2, None)
        for q in range(2):
            @pl.when(g2 < NGROUP - 1)
            def _():
                pltpu.make_async_copy(in_hbm.at[b0, 0, pl.ds(off, HB // 2), :],
                                      sh.at[sid, q], sem_sh_in[q]).wait()
                pltpu.async_copy(sh.at[sid, q],
                                 out_hbm.at[b0, 1, pl.ds(off, HB // 2), :],
                                 sem_sh_out[q])
        return 0

    lax.fori_loop(0, NGROUP, group2, 0)

    for b in range(NBUF):
        pltpu.make_async_copy(bufs_out[b], out_hbm.at[b0, 0, pl.ds(0, HB), :],
                              sem_out[b]).wait()


_sc_kernel = pl.kernel(
    _sc_body,
    out_type=jax.ShapeDtypeStruct((B, C, H, W), jnp.float32),
    mesh=plsc.VectorSubcoreMesh(core_axis_name="c", subcore_axis_name="s",
                                num_cores=NC, num_subcores=NS),
    scratch_types=(
        [pltpu.VMEM((2 * ROWS_PER_W * L,), jnp.float32)]
        + [pltpu.VMEM((HB, W), jnp.float32) for _ in range(2 * NBUF)]
        + [pltpu.SemaphoreType.DMA for _ in range(2 * NBUF + 1)]
        + [pltpu.VMEM_SHARED((NS, 2, HB // 2, W), jnp.float32)]
        + [pltpu.SemaphoreType.DMA for _ in range(4)]
    ),
    compiler_params=pltpu.CompilerParams(use_tc_tiling_on_sc=True),
)


def kernel(input, probs, magnitudes):
    p_b = jnp.broadcast_to(probs.astype(jnp.float32)[:, None], (B, L)).reshape(B * L)
    m_b = jnp.broadcast_to(magnitudes[:, None], (B, L)).reshape(B * L)
    return _sc_kernel(input, p_b, m_b)


# speedup vs baseline: 1.2760x; 1.2760x over previous
import jax
import jax.numpy as jnp
from jax import lax
from jax.experimental import pallas as pl
from jax.experimental.pallas import tpu as pltpu
from jax.experimental.pallas import tpu_sc as plsc

B, C, H, W = 64, 3, 512, 512
L = 16
NC, NS = 2, 16
NWORK = NC * NS
ROWS_PER_W = B // NWORK
NBUF = 3
HB = 32
CPP = H // HB
CPR = C * CPP
NCHUNK = ROWS_PER_W * CPR
NGROUP = NCHUNK // NBUF


def _sc_body(in_hbm, p_hbm, m_hbm, out_hbm, pm_v, *scratch):
    bufs_in = scratch[:NBUF]
    bufs_out = scratch[NBUF:2 * NBUF]
    sem_in = scratch[2 * NBUF:3 * NBUF]
    sem_out = scratch[3 * NBUF:4 * NBUF]
    sem_pm = scratch[4 * NBUF]

    wid = lax.axis_index("s") * NC + lax.axis_index("c")
    b0 = wid * ROWS_PER_W

    pltpu.async_copy(p_hbm.at[pl.ds(wid * (ROWS_PER_W * L), ROWS_PER_W * L)],
                     pm_v.at[pl.ds(0, ROWS_PER_W * L)], sem_pm)
    pltpu.async_copy(m_hbm.at[pl.ds(wid * (ROWS_PER_W * L), ROWS_PER_W * L)],
                     pm_v.at[pl.ds(ROWS_PER_W * L, ROWS_PER_W * L)], sem_pm)

    def chunk_coords(k):
        r = k // CPR
        rem = k - r * CPR
        c = rem // CPP
        h0 = (rem - c * CPP) * HB
        return b0 + r, r, c, h0

    for b in range(NBUF):
        pltpu.async_copy(in_hbm.at[b0, 0, pl.ds(b * HB, HB), :],
                         bufs_in[b], sem_in[b])

    pltpu.make_async_copy(p_hbm.at[pl.ds(0, ROWS_PER_W * L)],
                          pm_v.at[pl.ds(0, ROWS_PER_W * L)], sem_pm).wait()
    pltpu.make_async_copy(p_hbm.at[pl.ds(0, ROWS_PER_W * L)],
                          pm_v.at[pl.ds(0, ROWS_PER_W * L)], sem_pm).wait()

    addends = []
    for r in range(ROWS_PER_W):
        pvec = pm_v[pl.ds(r * L, L)]
        mvec = pm_v[pl.ds(ROWS_PER_W * L + r * L, L)]
        addends.append(jnp.where(pvec != 0.0, mvec, 0.0))

    def group(g, _):
        for b in range(NBUF):
            k = g * NBUF + b
            bi, r, c, h0 = chunk_coords(k)

            pltpu.make_async_copy(in_hbm.at[bi, c, pl.ds(h0, HB), :],
                                  bufs_in[b], sem_in[b]).wait()

            @pl.when(g >= 1)
            def _():
                pltpu.make_async_copy(
                    bufs_out[b], out_hbm.at[b0, 0, pl.ds(0, HB), :],
                    sem_out[b]).wait()

            addend = jnp.where(r == 0, addends[0], addends[1])

            @plsc.parallel_loop(0, HB)
            def _(i):
                for j in range(W // L):
                    sl = pl.ds(j * L, L)
                    bufs_out[b][i, sl] = bufs_in[b][i, sl] + addend

            pltpu.async_copy(bufs_out[b], out_hbm.at[bi, c, pl.ds(h0, HB), :],
                             sem_out[b])

            @pl.when(g < NGROUP - 1)
            def _():
                bn, _, cn, hn = chunk_coords(k + NBUF)
                pltpu.async_copy(in_hbm.at[bn, cn, pl.ds(hn, HB), :],
                                 bufs_in[b], sem_in[b])
        return 0

    lax.fori_loop(0, NGROUP, group, 0)

    for b in range(NBUF):
        pltpu.make_async_copy(bufs_out[b], out_hbm.at[b0, 0, pl.ds(0, HB), :],
                              sem_out[b]).wait()


_sc_kernel = pl.kernel(
    _sc_body,
    out_type=jax.ShapeDtypeStruct((B, C, H, W), jnp.float32),
    mesh=plsc.VectorSubcoreMesh(core_axis_name="c", subcore_axis_name="s",
                                num_cores=NC, num_subcores=NS),
    scratch_types=(
        [pltpu.VMEM((2 * ROWS_PER_W * L,), jnp.float32)]
        + [pltpu.VMEM((HB, W), jnp.float32) for _ in range(2 * NBUF)]
        + [pltpu.SemaphoreType.DMA for _ in range(2 * NBUF + 1)]
    ),
    compiler_params=pltpu.CompilerParams(use_tc_tiling_on_sc=True),
)


def kernel(input, probs, magnitudes):
    p_b = jnp.broadcast_to(probs.astype(jnp.float32)[:, None], (B, L)).reshape(B * L)
    m_b = jnp.broadcast_to(magnitudes[:, None], (B, L)).reshape(B * L)
    return _sc_kernel(input, p_b, m_b)
